# speedup vs baseline: 2.4570x; 2.4570x over previous
import jax
import jax.numpy as jnp
from jax.experimental import pallas as pl
from jax.experimental.pallas import tpu as pltpu
from jax.experimental.pallas import tpu_sc as plsc

_B, _L, _D_IN = 4, 1024, 512
_G, _V = 2, 320
_D_G = 128
_N = _B * _L
_R = 512
_NSTEPS = _N // _R
_GATHER_WINDOW = 128


def _tc_body(x_ref, w_ref, b_ref, g_ref, idx_ref, perp_ref, acc_ref):
    i = pl.program_id(0)
    x = x_ref[...]
    idx_cols = []
    col_sums = []
    for g in range(_G):
        h = jnp.dot(x, w_ref[g], preferred_element_type=jnp.float32)
        h = h + b_ref[g]
        z = h + g_ref[g]
        idx_cols.append(jnp.argmax(z, axis=-1).astype(jnp.int32) + g * _V)
        m = jnp.max(h, axis=-1, keepdims=True)
        e = jnp.exp(h - m)
        p = e / jnp.sum(e, axis=-1, keepdims=True)
        col_sums.append(jnp.sum(p, axis=0))
    idx_ref[...] = jnp.concatenate([c[:, None] for c in idx_cols], axis=1)
    colsum = jnp.stack(col_sums, axis=0)

    @pl.when(i == 0)
    def _():
        acc_ref[...] = colsum

    @pl.when(i > 0)
    def _():
        acc_ref[...] = acc_ref[...] + colsum

    @pl.when(i == _NSTEPS - 1)
    def _():
        avg = acc_ref[...] * (1.0 / _N)
        t = jnp.sum(avg * jnp.log(avg + 1e-7), axis=-1)
        perp_ref[0, 0] = jnp.sum(jnp.exp(-t))


def _tc_call(x2, wt, bt, gt):
    return pl.pallas_call(
        _tc_body,
        grid=(_NSTEPS,),
        in_specs=[
            pl.BlockSpec((_R, _D_IN), lambda i: (i, 0)),
            pl.BlockSpec((_G, _D_IN, _V), lambda i: (0, 0, 0)),
            pl.BlockSpec((_G, 1, _V), lambda i: (0, 0, 0)),
            pl.BlockSpec((_G, _R, _V), lambda i: (0, i, 0)),
        ],
        out_specs=[
            pl.BlockSpec((_R, _G), lambda i: (i, 0)),
            pl.BlockSpec(memory_space=pltpu.SMEM),
        ],
        out_shape=[
            jax.ShapeDtypeStruct((_N, _G), jnp.int32),
            jax.ShapeDtypeStruct((1, 1), jnp.float32),
        ],
        scratch_shapes=[pltpu.VMEM((_G, _V), jnp.float32)],
    )(x2, wt, bt, gt)


def _sc_gather(cb_flat, idx_flat):
    mesh = plsc.VectorSubcoreMesh(core_axis_name="core", subcore_axis_name="subcore")

    @pl.kernel(out_type=jax.ShapeDtypeStruct((_N * _G, _D_G), jnp.float32),
               mesh=mesh)
    def _gather_kernel(cb_hbm, i_hbm, o_hbm):
        def body(i_vmem, o_vmem):
            pltpu.sync_copy(cb_hbm.at[i_vmem.at[0]], o_vmem)

        pltpu.emit_pipeline(
            body,
            grid=((_N * _G) // _GATHER_WINDOW,),
            in_specs=[pl.BlockSpec((1, _GATHER_WINDOW), index_map=lambda i: (0, i))],
            out_specs=[pl.BlockSpec((_GATHER_WINDOW, _D_G), index_map=lambda i: (i, 0))],
            core_axis_name=("core", "subcore"),
            dimension_semantics=(pltpu.PARALLEL,),
        )(i_hbm, o_hbm)

    return _gather_kernel(cb_flat, idx_flat)


def kernel(hidden_states, W, b, codevectors):
    x2 = hidden_states.reshape(_N, _D_IN)
    wt = W.reshape(_D_IN, _G, _V).transpose(1, 0, 2)
    bt = b.reshape(_G, _V)[:, None, :]
    gum = jax.random.gumbel(jax.random.key(42), (_N * _G, _V), dtype=jnp.float32)
    gt = gum.reshape(_N, _G, _V).transpose(1, 0, 2)

    idx2, perp = _tc_call(x2, wt, bt, gt)

    cb_flat = codevectors.reshape(_G * _V, _D_G)
    idx_flat = idx2.reshape(1, _N * _G)
    cv_rows = _sc_gather(cb_flat, idx_flat)
    cv = cv_rows.reshape(_B, _L, _G * _D_G)
    return cv, perp.reshape(())


# speedup vs baseline: 3.2059x; 1.3048x over previous
import numpy as np

import jax
import jax.numpy as jnp
from jax.experimental import pallas as pl
from jax.experimental.pallas import tpu as pltpu
from jax.experimental.pallas import tpu_sc as plsc

_B, _L, _D_IN = 4, 1024, 512
_G, _V = 2, 320
_D_G = 128
_N = _B * _L
_R = 512
_NSTEPS = _N // _R
_GATHER_WINDOW = 128

_U32 = jnp.uint32


def _rotl(x, r):
    return jax.lax.shift_left(x, _U32(r)) | jax.lax.shift_right_logical(x, _U32(32 - r))


def _threefry2x32(k1, k2, x0, x1):
    rot = ((13, 15, 26, 6), (17, 29, 16, 24))
    ks = (k1, k2, k1 ^ k2 ^ _U32(0x1BD11BDA))
    x0 = x0 + ks[0]
    x1 = x1 + ks[1]
    sched = ((ks[1], ks[2], 1), (ks[2], ks[0], 2), (ks[0], ks[1], 3),
             (ks[1], ks[2], 4), (ks[2], ks[0], 5))
    for i, (a0, a1, c) in enumerate(sched):
        for r in rot[i % 2]:
            x0 = x0 + x1
            x1 = _rotl(x1, r)
            x1 = x1 ^ x0
        x0 = x0 + a0
        x1 = x1 + a1 + _U32(c)
    return x0, x1


def _gumbel_block(base, shape):
    rows, cols = shape
    row = jax.lax.broadcasted_iota(_U32, shape, 0)
    col = jax.lax.broadcasted_iota(_U32, shape, 1)
    c = base + row * _U32(_G * _V) + col
    b1, b2 = _threefry2x32(_U32(0), _U32(42), jnp.zeros(shape, _U32), c)
    bits = b1 ^ b2
    fb = jax.lax.shift_right_logical(bits, _U32(9)) | _U32(0x3F800000)
    floats = jax.lax.bitcast_convert_type(fb, jnp.float32) - jnp.float32(1.0)
    tiny = jnp.float32(np.finfo(np.float32).tiny)
    u = jnp.maximum(tiny, floats * (jnp.float32(1.0) - tiny) + tiny)
    return -jnp.log(-jnp.log(u))


def _tc_body(x_ref, w_ref, b_ref, idx_ref, perp_ref, acc_ref):
    i = pl.program_id(0)
    x = x_ref[...]
    idx_cols = []
    col_sums = []
    for g in range(_G):
        h = jnp.dot(x, w_ref[g], preferred_element_type=jnp.float32)
        h = h + b_ref[g]
        base = (_U32(i) * _U32(_R) * _U32(_G) + _U32(g)) * _U32(_V)
        z = h + _gumbel_block(base, (_R, _V))
        idx_cols.append(jnp.argmax(z, axis=-1).astype(jnp.int32) + g * _V)
        m = jnp.max(h, axis=-1, keepdims=True)
        e = jnp.exp(h - m)
        p = e / jnp.sum(e, axis=-1, keepdims=True)
        col_sums.append(jnp.sum(p, axis=0))
    idx_ref[...] = jnp.concatenate([c[:, None] for c in idx_cols], axis=1)
    colsum = jnp.stack(col_sums, axis=0)

    @pl.when(i == 0)
    def _():
        acc_ref[...] = colsum

    @pl.when(i > 0)
    def _():
        acc_ref[...] = acc_ref[...] + colsum

    @pl.when(i == _NSTEPS - 1)
    def _():
        avg = acc_ref[...] * (1.0 / _N)
        t = jnp.sum(avg * jnp.log(avg + 1e-7), axis=-1)
        perp_ref[0, 0] = jnp.sum(jnp.exp(-t))


def _tc_call(x2, wt, bt):
    return pl.pallas_call(
        _tc_body,
        grid=(_NSTEPS,),
        in_specs=[
            pl.BlockSpec((_R, _D_IN), lambda i: (i, 0)),
            pl.BlockSpec((_G, _D_IN, _V), lambda i: (0, 0, 0)),
            pl.BlockSpec((_G, 1, _V), lambda i: (0, 0, 0)),
        ],
        out_specs=[
            pl.BlockSpec((_R, _G), lambda i: (i, 0)),
            pl.BlockSpec(memory_space=pltpu.SMEM),
        ],
        out_shape=[
            jax.ShapeDtypeStruct((_N, _G), jnp.int32),
            jax.ShapeDtypeStruct((1, 1), jnp.float32),
        ],
        scratch_shapes=[pltpu.VMEM((_G, _V), jnp.float32)],
    )(x2, wt, bt)


def _sc_gather(cb_flat, idx_flat):
    mesh = plsc.VectorSubcoreMesh(core_axis_name="core", subcore_axis_name="subcore")

    @pl.kernel(out_type=jax.ShapeDtypeStruct((_N * _G, _D_G), jnp.float32),
               mesh=mesh)
    def _gather_kernel(cb_hbm, i_hbm, o_hbm):
        def body(i_vmem, o_vmem):
            pltpu.sync_copy(cb_hbm.at[i_vmem.at[0]], o_vmem)

        pltpu.emit_pipeline(
            body,
            grid=((_N * _G) // _GATHER_WINDOW,),
            in_specs=[pl.BlockSpec((1, _GATHER_WINDOW), index_map=lambda i: (0, i))],
            out_specs=[pl.BlockSpec((_GATHER_WINDOW, _D_G), index_map=lambda i: (i, 0))],
            core_axis_name=("core", "subcore"),
            dimension_semantics=(pltpu.PARALLEL,),
        )(i_hbm, o_hbm)

    return _gather_kernel(cb_flat, idx_flat)


def kernel(hidden_states, W, b, codevectors):
    x2 = hidden_states.reshape(_N, _D_IN)
    wt = W.reshape(_D_IN, _G, _V).transpose(1, 0, 2)
    bt = b.reshape(_G, _V)[:, None, :]

    idx2, perp = _tc_call(x2, wt, bt)

    cb_flat = codevectors.reshape(_G * _V, _D_G)
    idx_flat = idx2.reshape(1, _N * _G)
    cv_rows = _sc_gather(cb_flat, idx_flat)
    cv = cv_rows.reshape(_B, _L, _G * _D_G)
    return cv, perp.reshape(())


# speedup vs baseline: 3.6340x; 1.1335x over previous
import numpy as np

import jax
import jax.numpy as jnp
from jax.experimental import pallas as pl
from jax.experimental.pallas import tpu as pltpu
from jax.experimental.pallas import tpu_sc as plsc

_B, _L, _D_IN = 4, 1024, 512
_G, _V = 2, 320
_D_G = 128
_N = _B * _L
_R = 512
_NSTEPS = _N // _R
_GATHER_WINDOW = 128

_U32 = jnp.uint32


def _rotl(x, r):
    return jax.lax.shift_left(x, _U32(r)) | jax.lax.shift_right_logical(x, _U32(32 - r))


def _threefry2x32(k1, k2, x0, x1):
    rot = ((13, 15, 26, 6), (17, 29, 16, 24))
    ks = (k1, k2, k1 ^ k2 ^ _U32(0x1BD11BDA))
    x0 = x0 + ks[0]
    x1 = x1 + ks[1]
    sched = ((ks[1], ks[2], 1), (ks[2], ks[0], 2), (ks[0], ks[1], 3),
             (ks[1], ks[2], 4), (ks[2], ks[0], 5))
    for i, (a0, a1, c) in enumerate(sched):
        for r in rot[i % 2]:
            x0 = x0 + x1
            x1 = _rotl(x1, r)
            x1 = x1 ^ x0
        x0 = x0 + a0
        x1 = x1 + a1 + _U32(c)
    return x0, x1


def _gumbel_block(base, shape):
    rows, cols = shape
    row = jax.lax.broadcasted_iota(_U32, shape, 0)
    col = jax.lax.broadcasted_iota(_U32, shape, 1)
    c = base + row * _U32(_G * _V) + col
    b1, b2 = _threefry2x32(_U32(0), _U32(42), jnp.zeros(shape, _U32), c)
    bits = b1 ^ b2
    fb = jax.lax.shift_right_logical(bits, _U32(9)) | _U32(0x3F800000)
    floats = jax.lax.bitcast_convert_type(fb, jnp.float32) - jnp.float32(1.0)
    tiny = jnp.float32(np.finfo(np.float32).tiny)
    u = jnp.maximum(tiny, floats * (jnp.float32(1.0) - tiny) + tiny)
    return -jnp.log(-jnp.log(u))


def _tc_body(x_ref, w_ref, b_ref, idx_ref, perp_ref, acc_ref):
    i = pl.program_id(0)
    x = x_ref[...]
    gv = _G * _V
    h = jnp.dot(x, w_ref[...], preferred_element_type=jnp.float32)
    h = h + b_ref[...]
    base = _U32(i) * _U32(_R * gv)
    z = h + _gumbel_block(base, (_R, gv))
    sel = jax.lax.broadcasted_iota(jnp.int32, (_R, gv), 1) < _V
    ninf = jnp.float32(-jnp.inf)
    i0 = jnp.argmax(jnp.where(sel, z, ninf), axis=-1).astype(jnp.int32)
    i1 = jnp.argmax(jnp.where(sel, ninf, z), axis=-1).astype(jnp.int32)
    idx_ref[...] = jnp.concatenate([i0[:, None], i1[:, None]], axis=1)
    m0 = jnp.max(jnp.where(sel, h, ninf), axis=-1, keepdims=True)
    m1 = jnp.max(jnp.where(sel, ninf, h), axis=-1, keepdims=True)
    e = jnp.exp(h - jnp.where(sel, m0, m1))
    s0 = jnp.sum(jnp.where(sel, e, 0.0), axis=-1, keepdims=True)
    s1 = jnp.sum(jnp.where(sel, 0.0, e), axis=-1, keepdims=True)
    p = e / jnp.where(sel, s0, s1)
    colsum = jnp.sum(p, axis=0)[None, :]

    @pl.when(i == 0)
    def _():
        acc_ref[...] = colsum

    @pl.when(i > 0)
    def _():
        acc_ref[...] = acc_ref[...] + colsum

    @pl.when(i == _NSTEPS - 1)
    def _():
        avg = acc_ref[...] * (1.0 / _N)
        q = avg * jnp.log(avg + 1e-7)
        sel1 = jax.lax.broadcasted_iota(jnp.int32, (1, gv), 1) < _V
        t0 = jnp.sum(jnp.where(sel1, q, 0.0))
        t1 = jnp.sum(jnp.where(sel1, 0.0, q))
        perp_ref[0, 0] = jnp.exp(-t0) + jnp.exp(-t1)


def _tc_call(x2, w, b2):
    return pl.pallas_call(
        _tc_body,
        grid=(_NSTEPS,),
        in_specs=[
            pl.BlockSpec((_R, _D_IN), lambda i: (i, 0)),
            pl.BlockSpec((_D_IN, _G * _V), lambda i: (0, 0)),
            pl.BlockSpec((1, _G * _V), lambda i: (0, 0)),
        ],
        out_specs=[
            pl.BlockSpec((_R, _G), lambda i: (i, 0)),
            pl.BlockSpec(memory_space=pltpu.SMEM),
        ],
        out_shape=[
            jax.ShapeDtypeStruct((_N, _G), jnp.int32),
            jax.ShapeDtypeStruct((1, 1), jnp.float32),
        ],
        scratch_shapes=[pltpu.VMEM((1, _G * _V), jnp.float32)],
    )(x2, w, b2)


def _sc_gather(cb_flat, idx_flat):
    mesh = plsc.VectorSubcoreMesh(core_axis_name="core", subcore_axis_name="subcore")

    @pl.kernel(out_type=jax.ShapeDtypeStruct((_N * _G, _D_G), jnp.float32),
               mesh=mesh)
    def _gather_kernel(cb_hbm, i_hbm, o_hbm):
        def body(i_vmem, o_vmem):
            pltpu.sync_copy(cb_hbm.at[i_vmem.at[0]], o_vmem)

        pltpu.emit_pipeline(
            body,
            grid=((_N * _G) // _GATHER_WINDOW,),
            in_specs=[pl.BlockSpec((1, _GATHER_WINDOW), index_map=lambda i: (0, i))],
            out_specs=[pl.BlockSpec((_GATHER_WINDOW, _D_G), index_map=lambda i: (i, 0))],
            core_axis_name=("core", "subcore"),
            dimension_semantics=(pltpu.PARALLEL,),
        )(i_hbm, o_hbm)

    return _gather_kernel(cb_flat, idx_flat)


def kernel(hidden_states, W, b, codevectors):
    x2 = hidden_states.reshape(_N, _D_IN)
    b2 = b.reshape(1, _G * _V)

    idx2, perp = _tc_call(x2, W, b2)

    cb_flat = codevectors.reshape(_G * _V, _D_G)
    idx_flat = idx2.reshape(1, _N * _G)
    cv_rows = _sc_gather(cb_flat, idx_flat)
    cv = cv_rows.reshape(_B, _L, _G * _D_G)
    return cv, perp.reshape(())


# speedup vs baseline: 3.6403x; 1.0017x over previous
import numpy as np

import jax
import jax.numpy as jnp
from jax.experimental import pallas as pl
from jax.experimental.pallas import tpu as pltpu
from jax.experimental.pallas import tpu_sc as plsc

_B, _L, _D_IN = 4, 1024, 512
_G, _V = 2, 320
_D_G = 128
_N = _B * _L
_R = 512
_NSTEPS = _N // _R
_GATHER_WINDOW = 128

_U32 = jnp.uint32


def _rotl(x, r):
    return jax.lax.shift_left(x, _U32(r)) | jax.lax.shift_right_logical(x, _U32(32 - r))


def _threefry2x32(k1, k2, x0, x1):
    rot = ((13, 15, 26, 6), (17, 29, 16, 24))
    ks = (k1, k2, k1 ^ k2 ^ _U32(0x1BD11BDA))
    x0 = x0 + ks[0]
    x1 = x1 + ks[1]
    sched = ((ks[1], ks[2], 1), (ks[2], ks[0], 2), (ks[0], ks[1], 3),
             (ks[1], ks[2], 4), (ks[2], ks[0], 5))
    for i, (a0, a1, c) in enumerate(sched):
        for r in rot[i % 2]:
            x0 = x0 + x1
            x1 = _rotl(x1, r)
            x1 = x1 ^ x0
        x0 = x0 + a0
        x1 = x1 + a1 + _U32(c)
    return x0, x1


def _gumbel_block(base, iota, shape):
    c = base + iota
    b1, b2 = _threefry2x32(_U32(0), _U32(42), jnp.zeros(shape, _U32), c)
    bits = b1 ^ b2
    fb = jax.lax.shift_right_logical(bits, _U32(9)) | _U32(0x3F800000)
    floats = jax.lax.bitcast_convert_type(fb, jnp.float32) - jnp.float32(1.0)
    tiny = jnp.float32(np.finfo(np.float32).tiny)
    u = jnp.maximum(tiny, floats * (jnp.float32(1.0) - tiny) + tiny)
    return -jnp.log(-jnp.log(u))


def _tc_body(x_ref, w_ref, b_ref, iota_ref, idx_ref, perp_ref, acc_ref):
    i = pl.program_id(0)
    x = x_ref[...]
    gv = _G * _V
    h = jnp.dot(x, w_ref[...], preferred_element_type=jnp.float32)
    h = h + b_ref[...]
    base = _U32(i) * _U32(_R * gv)
    z = h + _gumbel_block(base, iota_ref[...], (_R, gv))
    sel = jax.lax.broadcasted_iota(jnp.int32, (_R, gv), 1) < _V
    ninf = jnp.float32(-jnp.inf)
    i0 = jnp.argmax(jnp.where(sel, z, ninf), axis=-1).astype(jnp.int32)
    i1 = jnp.argmax(jnp.where(sel, ninf, z), axis=-1).astype(jnp.int32)
    idx_ref[...] = jnp.concatenate([i0[:, None], i1[:, None]], axis=1)
    m0 = jnp.max(jnp.where(sel, h, ninf), axis=-1, keepdims=True)
    m1 = jnp.max(jnp.where(sel, ninf, h), axis=-1, keepdims=True)
    e = jnp.exp(h - jnp.where(sel, m0, m1))
    s0 = jnp.sum(jnp.where(sel, e, 0.0), axis=-1, keepdims=True)
    s1 = jnp.sum(jnp.where(sel, 0.0, e), axis=-1, keepdims=True)
    p = e / jnp.where(sel, s0, s1)
    colsum = jnp.sum(p, axis=0)[None, :]

    @pl.when(i == 0)
    def _():
        acc_ref[...] = colsum

    @pl.when(i > 0)
    def _():
        acc_ref[...] = acc_ref[...] + colsum

    @pl.when(i == _NSTEPS - 1)
    def _():
        avg = acc_ref[...] * (1.0 / _N)
        q = avg * jnp.log(avg + 1e-7)
        sel1 = jax.lax.broadcasted_iota(jnp.int32, (1, gv), 1) < _V
        t0 = jnp.sum(jnp.where(sel1, q, 0.0))
        t1 = jnp.sum(jnp.where(sel1, 0.0, q))
        perp_ref[0, 0] = jnp.exp(-t0) + jnp.exp(-t1)


def _tc_call(x2, w, b2):
    iota = (np.arange(_R, dtype=np.uint32)[:, None] * np.uint32(_G * _V)
            + np.arange(_G * _V, dtype=np.uint32)[None, :])
    return pl.pallas_call(
        _tc_body,
        grid=(_NSTEPS,),
        in_specs=[
            pl.BlockSpec((_R, _D_IN), lambda i: (i, 0)),
            pl.BlockSpec((_D_IN, _G * _V), lambda i: (0, 0)),
            pl.BlockSpec((1, _G * _V), lambda i: (0, 0)),
            pl.BlockSpec((_R, _G * _V), lambda i: (0, 0)),
        ],
        out_specs=[
            pl.BlockSpec((_R, _G), lambda i: (i, 0)),
            pl.BlockSpec(memory_space=pltpu.SMEM),
        ],
        out_shape=[
            jax.ShapeDtypeStruct((_N, _G), jnp.int32),
            jax.ShapeDtypeStruct((1, 1), jnp.float32),
        ],
        scratch_shapes=[pltpu.VMEM((1, _G * _V), jnp.float32)],
    )(x2, w, b2, jnp.asarray(iota))


def _sc_gather(cb_flat, idx_flat):
    mesh = plsc.VectorSubcoreMesh(core_axis_name="core", subcore_axis_name="subcore")

    @pl.kernel(out_type=jax.ShapeDtypeStruct((_N * _G, _D_G), jnp.float32),
               mesh=mesh)
    def _gather_kernel(cb_hbm, i_hbm, o_hbm):
        def body(i_vmem, o_vmem):
            pltpu.sync_copy(cb_hbm.at[i_vmem.at[0]], o_vmem)

        pltpu.emit_pipeline(
            body,
            grid=((_N * _G) // _GATHER_WINDOW,),
            in_specs=[pl.BlockSpec((1, _GATHER_WINDOW), index_map=lambda i: (0, i))],
            out_specs=[pl.BlockSpec((_GATHER_WINDOW, _D_G), index_map=lambda i: (i, 0))],
            core_axis_name=("core", "subcore"),
            dimension_semantics=(pltpu.PARALLEL,),
        )(i_hbm, o_hbm)

    return _gather_kernel(cb_flat, idx_flat)


def kernel(hidden_states, W, b, codevectors):
    x2 = hidden_states.reshape(_N, _D_IN)
    b2 = b.reshape(1, _G * _V)

    idx2, perp = _tc_call(x2, W, b2)

    cb_flat = codevectors.reshape(_G * _V, _D_G)
    idx_flat = idx2.reshape(1, _N * _G)
    cv_rows = _sc_gather(cb_flat, idx_flat)
    cv = cv_rows.reshape(_B, _L, _G * _D_G)
    return cv, perp.reshape(())


# speedup vs baseline: 4.1385x; 1.1369x over previous
import numpy as np

import jax
import jax.numpy as jnp
from jax.experimental import pallas as pl
from jax.experimental.pallas import tpu as pltpu
from jax.experimental.pallas import tpu_sc as plsc

_B, _L, _D_IN = 4, 1024, 512
_G, _V = 2, 320
_D_G = 128
_N = _B * _L
_R = 512
_NSTEPS = _N // _R
_GATHER_WINDOW = 64

_U32 = jnp.uint32


def _rotl(x, r):
    return jax.lax.shift_left(x, _U32(r)) | jax.lax.shift_right_logical(x, _U32(32 - r))


def _threefry2x32(k1, k2, x0, x1):
    rot = ((13, 15, 26, 6), (17, 29, 16, 24))
    ks = (k1, k2, k1 ^ k2 ^ _U32(0x1BD11BDA))
    x0 = x0 + ks[0]
    x1 = x1 + ks[1]
    sched = ((ks[1], ks[2], 1), (ks[2], ks[0], 2), (ks[0], ks[1], 3),
             (ks[1], ks[2], 4), (ks[2], ks[0], 5))
    for i, (a0, a1, c) in enumerate(sched):
        for r in rot[i % 2]:
            x0 = x0 + x1
            x1 = _rotl(x1, r)
            x1 = x1 ^ x0
        x0 = x0 + a0
        x1 = x1 + a1 + _U32(c)
    return x0, x1


def _gumbel_block(base, iota, shape):
    c = base + iota
    b1, b2 = _threefry2x32(_U32(0), _U32(42), jnp.zeros(shape, _U32), c)
    bits = b1 ^ b2
    fb = jax.lax.shift_right_logical(bits, _U32(9)) | _U32(0x3F800000)
    floats = jax.lax.bitcast_convert_type(fb, jnp.float32) - jnp.float32(1.0)
    tiny = jnp.float32(np.finfo(np.float32).tiny)
    u = jnp.maximum(tiny, floats * (jnp.float32(1.0) - tiny) + tiny)
    return -jnp.log(-jnp.log(u))


def _tc_body(x_ref, w_ref, b_ref, iota_ref, idx0_ref, idx1_ref, perp_ref, acc_ref):
    i = pl.program_id(0)
    x = x_ref[...]
    gv = _G * _V
    h = jnp.dot(x, w_ref[...], preferred_element_type=jnp.float32)
    h = h + b_ref[...]
    base = _U32(i) * _U32(_R * gv)
    z = h + _gumbel_block(base, iota_ref[...], (_R, gv))
    sel = jax.lax.broadcasted_iota(jnp.int32, (_R, gv), 1) < _V
    ninf = jnp.float32(-jnp.inf)
    i0 = jnp.argmax(jnp.where(sel, z, ninf), axis=-1).astype(jnp.int32)
    i1 = jnp.argmax(jnp.where(sel, ninf, z), axis=-1).astype(jnp.int32)
    idx0_ref[...] = i0.reshape(1, _R)
    idx1_ref[...] = i1.reshape(1, _R)
    m0 = jnp.max(jnp.where(sel, h, ninf), axis=-1, keepdims=True)
    m1 = jnp.max(jnp.where(sel, ninf, h), axis=-1, keepdims=True)
    e = jnp.exp(h - jnp.where(sel, m0, m1))
    s0 = jnp.sum(jnp.where(sel, e, 0.0), axis=-1, keepdims=True)
    s1 = jnp.sum(jnp.where(sel, 0.0, e), axis=-1, keepdims=True)
    p = e / jnp.where(sel, s0, s1)
    colsum = jnp.sum(p, axis=0)[None, :]

    @pl.when(i == 0)
    def _():
        acc_ref[...] = colsum

    @pl.when(i > 0)
    def _():
        acc_ref[...] = acc_ref[...] + colsum

    @pl.when(i == _NSTEPS - 1)
    def _():
        avg = acc_ref[...] * (1.0 / _N)
        q = avg * jnp.log(avg + 1e-7)
        sel1 = jax.lax.broadcasted_iota(jnp.int32, (1, gv), 1) < _V
        t0 = jnp.sum(jnp.where(sel1, q, 0.0))
        t1 = jnp.sum(jnp.where(sel1, 0.0, q))
        perp_ref[0, 0] = jnp.exp(-t0) + jnp.exp(-t1)


def _tc_call(x2, w, b2):
    iota = (np.arange(_R, dtype=np.uint32)[:, None] * np.uint32(_G * _V)
            + np.arange(_G * _V, dtype=np.uint32)[None, :])
    return pl.pallas_call(
        _tc_body,
        grid=(_NSTEPS,),
        in_specs=[
            pl.BlockSpec((_R, _D_IN), lambda i: (i, 0)),
            pl.BlockSpec((_D_IN, _G * _V), lambda i: (0, 0)),
            pl.BlockSpec((1, _G * _V), lambda i: (0, 0)),
            pl.BlockSpec((_R, _G * _V), lambda i: (0, 0)),
        ],
        out_specs=[
            pl.BlockSpec((1, _R), lambda i: (0, i)),
            pl.BlockSpec((1, _R), lambda i: (0, i)),
            pl.BlockSpec(memory_space=pltpu.SMEM),
        ],
        out_shape=[
            jax.ShapeDtypeStruct((1, _N), jnp.int32),
            jax.ShapeDtypeStruct((1, _N), jnp.int32),
            jax.ShapeDtypeStruct((1, 1), jnp.float32),
        ],
        scratch_shapes=[pltpu.VMEM((1, _G * _V), jnp.float32)],
    )(x2, w, b2, jnp.asarray(iota))


def _sc_gather(cb_flat, idx0, idx1):
    mesh = plsc.VectorSubcoreMesh(core_axis_name="core", subcore_axis_name="subcore")

    @pl.kernel(out_type=jax.ShapeDtypeStruct((_N, _G * _D_G), jnp.float32),
               mesh=mesh)
    def _gather_kernel(cb_hbm, i0_hbm, i1_hbm, o_hbm):
        def make_body(lo):
            def body(i0_vmem, i1_vmem, o_vmem):
                w = _GATHER_WINDOW
                pltpu.sync_copy(cb_hbm.at[i0_vmem.at[0, pl.ds(lo, w)]],
                                o_vmem.at[:, :_D_G])
                pltpu.sync_copy(cb_hbm.at[i1_vmem.at[0, pl.ds(lo, w)]],
                                o_vmem.at[:, _D_G:])
            return body

        for half in (0, 1):
            pltpu.emit_pipeline(
                make_body(half * _GATHER_WINDOW),
                grid=(_N // (2 * _GATHER_WINDOW),),
                in_specs=[pl.BlockSpec((1, 2 * _GATHER_WINDOW),
                                       index_map=lambda i: (0, i)),
                          pl.BlockSpec((1, 2 * _GATHER_WINDOW),
                                       index_map=lambda i: (0, i))],
                out_specs=[pl.BlockSpec(
                    (_GATHER_WINDOW, _G * _D_G),
                    index_map=lambda i, h=half: (2 * i + h, 0))],
                core_axis_name=("core", "subcore"),
                dimension_semantics=(pltpu.PARALLEL,),
            )(i0_hbm, i1_hbm, o_hbm)

    return _gather_kernel(cb_flat, idx0, idx1)


def kernel(hidden_states, W, b, codevectors):
    x2 = hidden_states.reshape(_N, _D_IN)
    b2 = b.reshape(1, _G * _V)

    idx0, idx1, perp = _tc_call(x2, W, b2)

    cb_flat = codevectors.reshape(_G * _V, _D_G)
    cv_rows = _sc_gather(cb_flat, idx0, idx1)
    cv = cv_rows.reshape(_B, _L, _G * _D_G)
    return cv, perp.reshape(())


# speedup vs baseline: 4.1425x; 1.0010x over previous
import numpy as np

import jax
import jax.numpy as jnp
from jax.experimental import pallas as pl
from jax.experimental.pallas import tpu as pltpu
from jax.experimental.pallas import tpu_sc as plsc

_B, _L, _D_IN = 4, 1024, 512
_G, _V = 2, 320
_D_G = 128
_N = _B * _L
_R = 512
_NSTEPS = _N // _R
_GATHER_WINDOW = 64

_U32 = jnp.uint32


def _rotl(x, r):
    return jax.lax.shift_left(x, _U32(r)) | jax.lax.shift_right_logical(x, _U32(32 - r))


def _threefry2x32(k1, k2, x0, x1):
    rot = ((13, 15, 26, 6), (17, 29, 16, 24))
    ks = (k1, k2, k1 ^ k2 ^ _U32(0x1BD11BDA))
    x0 = x0 + ks[0]
    x1 = x1 + ks[1]
    sched = ((ks[1], ks[2], 1), (ks[2], ks[0], 2), (ks[0], ks[1], 3),
             (ks[1], ks[2], 4), (ks[2], ks[0], 5))
    for i, (a0, a1, c) in enumerate(sched):
        for r in rot[i % 2]:
            x0 = x0 + x1
            x1 = _rotl(x1, r)
            x1 = x1 ^ x0
        x0 = x0 + a0
        x1 = x1 + a1 + _U32(c)
    return x0, x1


def _gumbel_block(base, shape):
    row = jax.lax.broadcasted_iota(_U32, shape, 0)
    col = jax.lax.broadcasted_iota(_U32, shape, 1)
    c = base + row * _U32(_G * _V) + col
    b1, b2 = _threefry2x32(_U32(0), _U32(42), jnp.zeros(shape, _U32), c)
    bits = b1 ^ b2
    fb = jax.lax.shift_right_logical(bits, _U32(9)) | _U32(0x3F800000)
    floats = jax.lax.bitcast_convert_type(fb, jnp.float32) - jnp.float32(1.0)
    tiny = jnp.float32(np.finfo(np.float32).tiny)
    u = jnp.maximum(tiny, floats * (jnp.float32(1.0) - tiny) + tiny)
    return -jnp.log(-jnp.log(u))


def _tc_body(x_ref, w_ref, b_ref, idx0_ref, idx1_ref, perp_ref, acc_ref):
    i = pl.program_id(0)
    x = x_ref[...]
    gv = _G * _V
    h = jnp.dot(x, w_ref[...], preferred_element_type=jnp.float32)
    h = h + b_ref[...]
    base = _U32(i) * _U32(_R * gv)
    z = h + _gumbel_block(base, (_R, gv))
    sel = jax.lax.broadcasted_iota(jnp.int32, (_R, gv), 1) < _V
    ninf = jnp.float32(-jnp.inf)
    i0 = jnp.argmax(jnp.where(sel, z, ninf), axis=-1).astype(jnp.int32)
    i1 = jnp.argmax(jnp.where(sel, ninf, z), axis=-1).astype(jnp.int32)
    idx0_ref[...] = i0.reshape(1, _R)
    idx1_ref[...] = i1.reshape(1, _R)
    m0 = jnp.max(jnp.where(sel, h, ninf), axis=-1, keepdims=True)
    m1 = jnp.max(jnp.where(sel, ninf, h), axis=-1, keepdims=True)
    e = jnp.exp(h - jnp.where(sel, m0, m1))
    s0 = jnp.sum(jnp.where(sel, e, 0.0), axis=-1, keepdims=True)
    s1 = jnp.sum(jnp.where(sel, 0.0, e), axis=-1, keepdims=True)
    p = e / jnp.where(sel, s0, s1)
    colsum = jnp.sum(p, axis=0)[None, :]

    @pl.when(i == 0)
    def _():
        acc_ref[...] = colsum

    @pl.when(i > 0)
    def _():
        acc_ref[...] = acc_ref[...] + colsum

    @pl.when(i == _NSTEPS - 1)
    def _():
        avg = acc_ref[...] * (1.0 / _N)
        q = avg * jnp.log(avg + 1e-7)
        sel1 = jax.lax.broadcasted_iota(jnp.int32, (1, gv), 1) < _V
        t0 = jnp.sum(jnp.where(sel1, q, 0.0))
        t1 = jnp.sum(jnp.where(sel1, 0.0, q))
        perp_ref[0, 0] = jnp.exp(-t0) + jnp.exp(-t1)


def _tc_call(x2, w, b2):
    return pl.pallas_call(
        _tc_body,
        grid=(_NSTEPS,),
        in_specs=[
            pl.BlockSpec((_R, _D_IN), lambda i: (i, 0)),
            pl.BlockSpec((_D_IN, _G * _V), lambda i: (0, 0)),
            pl.BlockSpec((1, _G * _V), lambda i: (0, 0)),
        ],
        out_specs=[
            pl.BlockSpec((1, _R), lambda i: (0, i)),
            pl.BlockSpec((1, _R), lambda i: (0, i)),
            pl.BlockSpec(memory_space=pltpu.SMEM),
        ],
        out_shape=[
            jax.ShapeDtypeStruct((1, _N), jnp.int32),
            jax.ShapeDtypeStruct((1, _N), jnp.int32),
            jax.ShapeDtypeStruct((1, 1), jnp.float32),
        ],
        scratch_shapes=[pltpu.VMEM((1, _G * _V), jnp.float32)],
    )(x2, w, b2)


def _sc_gather(cb_flat, idx0, idx1):
    mesh = plsc.VectorSubcoreMesh(core_axis_name="core", subcore_axis_name="subcore")

    @pl.kernel(out_type=jax.ShapeDtypeStruct((_N, _G * _D_G), jnp.float32),
               mesh=mesh)
    def _gather_kernel(cb_hbm, i0_hbm, i1_hbm, o_hbm):
        def make_body(lo):
            def body(i0_vmem, i1_vmem, o_vmem):
                w = _GATHER_WINDOW
                pltpu.sync_copy(cb_hbm.at[i0_vmem.at[0, pl.ds(lo, w)]],
                                o_vmem.at[:, :_D_G])
                pltpu.sync_copy(cb_hbm.at[i1_vmem.at[0, pl.ds(lo, w)]],
                                o_vmem.at[:, _D_G:])
            return body

        for half in (0, 1):
            pltpu.emit_pipeline(
                make_body(half * _GATHER_WINDOW),
                grid=(_N // (2 * _GATHER_WINDOW),),
                in_specs=[pl.BlockSpec((1, 2 * _GATHER_WINDOW),
                                       index_map=lambda i: (0, i)),
                          pl.BlockSpec((1, 2 * _GATHER_WINDOW),
                                       index_map=lambda i: (0, i))],
                out_specs=[pl.BlockSpec(
                    (_GATHER_WINDOW, _G * _D_G),
                    index_map=lambda i, h=half: (2 * i + h, 0))],
                core_axis_name=("core", "subcore"),
                dimension_semantics=(pltpu.PARALLEL,),
            )(i0_hbm, i1_hbm, o_hbm)

    return _gather_kernel(cb_flat, idx0, idx1)


def kernel(hidden_states, W, b, codevectors):
    x2 = hidden_states.reshape(_N, _D_IN)
    b2 = b.reshape(1, _G * _V)

    idx0, idx1, perp = _tc_call(x2, W, b2)

    cb_flat = codevectors.reshape(_G * _V, _D_G)
    cv_rows = _sc_gather(cb_flat, idx0, idx1)
    cv = cv_rows.reshape(_B, _L, _G * _D_G)
    return cv, perp.reshape(())


# speedup vs baseline: 4.1748x; 1.0078x over previous
import numpy as np

import jax
import jax.numpy as jnp
from jax.experimental import pallas as pl
from jax.experimental.pallas import tpu as pltpu
from jax.experimental.pallas import tpu_sc as plsc

_B, _L, _D_IN = 4, 1024, 512
_G, _V = 2, 320
_D_G = 128
_N = _B * _L
_R = 1024
_NSTEPS = _N // _R
_GATHER_WINDOW = 64

_U32 = jnp.uint32


def _rotl(x, r):
    return jax.lax.shift_left(x, _U32(r)) | jax.lax.shift_right_logical(x, _U32(32 - r))


def _threefry2x32(k1, k2, x0, x1):
    rot = ((13, 15, 26, 6), (17, 29, 16, 24))
    ks = (k1, k2, k1 ^ k2 ^ _U32(0x1BD11BDA))
    x0 = x0 + ks[0]
    x1 = x1 + ks[1]
    sched = ((ks[1], ks[2], 1), (ks[2], ks[0], 2), (ks[0], ks[1], 3),
             (ks[1], ks[2], 4), (ks[2], ks[0], 5))
    for i, (a0, a1, c) in enumerate(sched):
        for r in rot[i % 2]:
            x0 = x0 + x1
            x1 = _rotl(x1, r)
            x1 = x1 ^ x0
        x0 = x0 + a0
        x1 = x1 + a1 + _U32(c)
    return x0, x1


def _gumbel_block(base, shape):
    row = jax.lax.broadcasted_iota(_U32, shape, 0)
    col = jax.lax.broadcasted_iota(_U32, shape, 1)
    c = base + row * _U32(_G * _V) + col
    b1, b2 = _threefry2x32(_U32(0), _U32(42), jnp.zeros(shape, _U32), c)
    bits = b1 ^ b2
    fb = jax.lax.shift_right_logical(bits, _U32(9)) | _U32(0x3F800000)
    floats = jax.lax.bitcast_convert_type(fb, jnp.float32) - jnp.float32(1.0)
    tiny = jnp.float32(np.finfo(np.float32).tiny)
    u = jnp.maximum(tiny, floats * (jnp.float32(1.0) - tiny) + tiny)
    return -jnp.log(-jnp.log(u))


def _tc_body(x_ref, w_ref, b_ref, idx0_ref, idx1_ref, perp_ref, acc_ref):
    i = pl.program_id(0)
    x = x_ref[...]
    gv = _G * _V
    h = jnp.dot(x, w_ref[...], preferred_element_type=jnp.float32)
    h = h + b_ref[...]
    base = _U32(i) * _U32(_R * gv)
    z = h + _gumbel_block(base, (_R, gv))
    sel = jax.lax.broadcasted_iota(jnp.int32, (_R, gv), 1) < _V
    ninf = jnp.float32(-jnp.inf)
    i0 = jnp.argmax(jnp.where(sel, z, ninf), axis=-1).astype(jnp.int32)
    i1 = jnp.argmax(jnp.where(sel, ninf, z), axis=-1).astype(jnp.int32)
    idx0_ref[...] = i0.reshape(1, _R)
    idx1_ref[...] = i1.reshape(1, _R)
    m0 = jnp.max(jnp.where(sel, h, ninf), axis=-1, keepdims=True)
    m1 = jnp.max(jnp.where(sel, ninf, h), axis=-1, keepdims=True)
    e = jnp.exp(h - jnp.where(sel, m0, m1))
    s0 = jnp.sum(jnp.where(sel, e, 0.0), axis=-1, keepdims=True)
    s1 = jnp.sum(jnp.where(sel, 0.0, e), axis=-1, keepdims=True)
    p = e / jnp.where(sel, s0, s1)
    colsum = jnp.sum(p, axis=0)[None, :]

    @pl.when(i == 0)
    def _():
        acc_ref[...] = colsum

    @pl.when(i > 0)
    def _():
        acc_ref[...] = acc_ref[...] + colsum

    @pl.when(i == _NSTEPS - 1)
    def _():
        avg = acc_ref[...] * (1.0 / _N)
        q = avg * jnp.log(avg + 1e-7)
        sel1 = jax.lax.broadcasted_iota(jnp.int32, (1, gv), 1) < _V
        t0 = jnp.sum(jnp.where(sel1, q, 0.0))
        t1 = jnp.sum(jnp.where(sel1, 0.0, q))
        perp_ref[0, 0] = jnp.exp(-t0) + jnp.exp(-t1)


def _tc_call(x2, w, b2):
    return pl.pallas_call(
        _tc_body,
        grid=(_NSTEPS,),
        in_specs=[
            pl.BlockSpec((_R, _D_IN), lambda i: (i, 0)),
            pl.BlockSpec((_D_IN, _G * _V), lambda i: (0, 0)),
            pl.BlockSpec((1, _G * _V), lambda i: (0, 0)),
        ],
        out_specs=[
            pl.BlockSpec((1, _R), lambda i: (0, i)),
            pl.BlockSpec((1, _R), lambda i: (0, i)),
            pl.BlockSpec(memory_space=pltpu.SMEM),
        ],
        out_shape=[
            jax.ShapeDtypeStruct((1, _N), jnp.int32),
            jax.ShapeDtypeStruct((1, _N), jnp.int32),
            jax.ShapeDtypeStruct((1, 1), jnp.float32),
        ],
        scratch_shapes=[pltpu.VMEM((1, _G * _V), jnp.float32)],
    )(x2, w, b2)


def _sc_gather(cb_flat, idx0, idx1):
    mesh = plsc.VectorSubcoreMesh(core_axis_name="core", subcore_axis_name="subcore")

    @pl.kernel(out_type=jax.ShapeDtypeStruct((_N, _G * _D_G), jnp.float32),
               mesh=mesh)
    def _gather_kernel(cb_hbm, i0_hbm, i1_hbm, o_hbm):
        def make_body(lo):
            def body(i0_vmem, i1_vmem, o_vmem):
                w = _GATHER_WINDOW
                pltpu.sync_copy(cb_hbm.at[i0_vmem.at[0, pl.ds(lo, w)]],
                                o_vmem.at[:, :_D_G])
                pltpu.sync_copy(cb_hbm.at[i1_vmem.at[0, pl.ds(lo, w)]],
                                o_vmem.at[:, _D_G:])
            return body

        for half in (0, 1):
            pltpu.emit_pipeline(
                make_body(half * _GATHER_WINDOW),
                grid=(_N // (2 * _GATHER_WINDOW),),
                in_specs=[pl.BlockSpec((1, 2 * _GATHER_WINDOW),
                                       index_map=lambda i: (0, i)),
                          pl.BlockSpec((1, 2 * _GATHER_WINDOW),
                                       index_map=lambda i: (0, i))],
                out_specs=[pl.BlockSpec(
                    (_GATHER_WINDOW, _G * _D_G),
                    index_map=lambda i, h=half: (2 * i + h, 0))],
                core_axis_name=("core", "subcore"),
                dimension_semantics=(pltpu.PARALLEL,),
            )(i0_hbm, i1_hbm, o_hbm)

    return _gather_kernel(cb_flat, idx0, idx1)


def kernel(hidden_states, W, b, codevectors):
    x2 = hidden_states.reshape(_N, _D_IN)
    b2 = b.reshape(1, _G * _V)

    idx0, idx1, perp = _tc_call(x2, W, b2)

    cb_flat = codevectors.reshape(_G * _V, _D_G)
    cv_rows = _sc_gather(cb_flat, idx0, idx1)
    cv = cv_rows.reshape(_B, _L, _G * _D_G)
    return cv, perp.reshape(())


# speedup vs baseline: 4.2242x; 1.0118x over previous
import numpy as np

import jax
import jax.numpy as jnp
from jax.experimental import pallas as pl
from jax.experimental.pallas import tpu as pltpu
from jax.experimental.pallas import tpu_sc as plsc

_B, _L, _D_IN = 4, 1024, 512
_G, _V = 2, 320
_D_G = 128
_N = _B * _L
_R = 1024
_NSTEPS = _N // _R
_GATHER_WINDOW = 128

_U32 = jnp.uint32


def _rotl(x, r):
    return jax.lax.shift_left(x, _U32(r)) | jax.lax.shift_right_logical(x, _U32(32 - r))


def _threefry2x32(k1, k2, x0, x1):
    rot = ((13, 15, 26, 6), (17, 29, 16, 24))
    ks = (k1, k2, k1 ^ k2 ^ _U32(0x1BD11BDA))
    x0 = x0 + ks[0]
    x1 = x1 + ks[1]
    sched = ((ks[1], ks[2], 1), (ks[2], ks[0], 2), (ks[0], ks[1], 3),
             (ks[1], ks[2], 4), (ks[2], ks[0], 5))
    for i, (a0, a1, c) in enumerate(sched):
        for r in rot[i % 2]:
            x0 = x0 + x1
            x1 = _rotl(x1, r)
            x1 = x1 ^ x0
        x0 = x0 + a0
        x1 = x1 + a1 + _U32(c)
    return x0, x1


def _gumbel_block(base, shape):
    row = jax.lax.broadcasted_iota(_U32, shape, 0)
    col = jax.lax.broadcasted_iota(_U32, shape, 1)
    c = base + row * _U32(_G * _V) + col
    b1, b2 = _threefry2x32(_U32(0), _U32(42), jnp.zeros(shape, _U32), c)
    bits = b1 ^ b2
    fb = jax.lax.shift_right_logical(bits, _U32(9)) | _U32(0x3F800000)
    floats = jax.lax.bitcast_convert_type(fb, jnp.float32) - jnp.float32(1.0)
    tiny = jnp.float32(np.finfo(np.float32).tiny)
    u = jnp.maximum(tiny, floats * (jnp.float32(1.0) - tiny) + tiny)
    return -jnp.log(-jnp.log(u))


def _tc_body(x_ref, w_ref, b_ref, idx0_ref, idx1_ref, perp_ref, acc_ref):
    i = pl.program_id(0)
    x = x_ref[...]
    gv = _G * _V
    h = jnp.dot(x, w_ref[...], preferred_element_type=jnp.float32)
    h = h + b_ref[...]
    base = _U32(i) * _U32(_R * gv)
    z = h + _gumbel_block(base, (_R, gv))
    sel = jax.lax.broadcasted_iota(jnp.int32, (_R, gv), 1) < _V
    ninf = jnp.float32(-jnp.inf)
    i0 = jnp.argmax(jnp.where(sel, z, ninf), axis=-1).astype(jnp.int32)
    i1 = jnp.argmax(jnp.where(sel, ninf, z), axis=-1).astype(jnp.int32)
    idx0_ref[...] = i0.reshape(1, _R)
    idx1_ref[...] = i1.reshape(1, _R)
    m0 = jnp.max(jnp.where(sel, h, ninf), axis=-1, keepdims=True)
    m1 = jnp.max(jnp.where(sel, ninf, h), axis=-1, keepdims=True)
    e = jnp.exp(h - jnp.where(sel, m0, m1))
    s0 = jnp.sum(jnp.where(sel, e, 0.0), axis=-1, keepdims=True)
    s1 = jnp.sum(jnp.where(sel, 0.0, e), axis=-1, keepdims=True)
    p = e / jnp.where(sel, s0, s1)
    colsum = jnp.sum(p, axis=0)[None, :]

    @pl.when(i == 0)
    def _():
        acc_ref[...] = colsum

    @pl.when(i > 0)
    def _():
        acc_ref[...] = acc_ref[...] + colsum

    @pl.when(i == _NSTEPS - 1)
    def _():
        avg = acc_ref[...] * (1.0 / _N)
        q = avg * jnp.log(avg + 1e-7)
        sel1 = jax.lax.broadcasted_iota(jnp.int32, (1, gv), 1) < _V
        t0 = jnp.sum(jnp.where(sel1, q, 0.0))
        t1 = jnp.sum(jnp.where(sel1, 0.0, q))
        perp_ref[0, 0] = jnp.exp(-t0) + jnp.exp(-t1)


def _tc_call(x2, w, b2):
    return pl.pallas_call(
        _tc_body,
        grid=(_NSTEPS,),
        in_specs=[
            pl.BlockSpec((_R, _D_IN), lambda i: (i, 0)),
            pl.BlockSpec((_D_IN, _G * _V), lambda i: (0, 0)),
            pl.BlockSpec((1, _G * _V), lambda i: (0, 0)),
        ],
        out_specs=[
            pl.BlockSpec((1, _R), lambda i: (0, i)),
            pl.BlockSpec((1, _R), lambda i: (0, i)),
            pl.BlockSpec(memory_space=pltpu.SMEM),
        ],
        out_shape=[
            jax.ShapeDtypeStruct((1, _N), jnp.int32),
            jax.ShapeDtypeStruct((1, _N), jnp.int32),
            jax.ShapeDtypeStruct((1, 1), jnp.float32),
        ],
        scratch_shapes=[pltpu.VMEM((1, _G * _V), jnp.float32)],
    )(x2, w, b2)


def _sc_gather(cb_flat, idx0, idx1):
    mesh = plsc.VectorSubcoreMesh(core_axis_name="core", subcore_axis_name="subcore")

    @pl.kernel(out_type=jax.ShapeDtypeStruct((_N, _G * _D_G), jnp.float32),
               mesh=mesh)
    def _gather_kernel(cb_hbm, i0_hbm, i1_hbm, o_hbm):
        def body(i0_vmem, i1_vmem, o_vmem):
            pltpu.sync_copy(cb_hbm.at[i0_vmem.at[0]], o_vmem.at[:, :_D_G])
            pltpu.sync_copy(cb_hbm.at[i1_vmem.at[0]], o_vmem.at[:, _D_G:])

        pltpu.emit_pipeline(
            body,
            grid=(_N // _GATHER_WINDOW,),
            in_specs=[pl.BlockSpec((1, _GATHER_WINDOW),
                                   index_map=lambda i: (0, i)),
                      pl.BlockSpec((1, _GATHER_WINDOW),
                                   index_map=lambda i: (0, i))],
            out_specs=[pl.BlockSpec((_GATHER_WINDOW, _G * _D_G),
                                    index_map=lambda i: (i, 0))],
            core_axis_name=("core", "subcore"),
            dimension_semantics=(pltpu.PARALLEL,),
        )(i0_hbm, i1_hbm, o_hbm)

    return _gather_kernel(cb_flat, idx0, idx1)


def kernel(hidden_states, W, b, codevectors):
    x2 = hidden_states.reshape(_N, _D_IN)
    b2 = b.reshape(1, _G * _V)

    idx0, idx1, perp = _tc_call(x2, W, b2)

    cb_flat = codevectors.reshape(_G * _V, _D_G)
    cv_rows = _sc_gather(cb_flat, idx0, idx1)
    cv = cv_rows.reshape(_B, _L, _G * _D_G)
    return cv, perp.reshape(())
